# Initial kernel scaffold; baseline (speedup 1.0000x reference)
import jax
import jax.numpy as jnp
from jax.experimental import pallas as pl


def kernel(x, edge_index, Wl1, Wr1, att1, b1, Wl2, Wr2, att2, b2, Wl3, Wr3, att3, b3):
    raise NotImplementedError("write your pallas kernel here")


# speedup vs baseline: 1.1388x; 1.1388x over previous
import functools

import jax
import jax.numpy as jnp
from jax.experimental import pallas as pl
from jax.experimental.pallas import tpu as pltpu

N = 10000
HEADS = 8
NHID = 16


def _mm2_body(x_ref, wl_ref, wr_ref, xl_ref, xr_ref):
    x = x_ref[...]
    xl_ref[...] = jnp.dot(x, wl_ref[...], preferred_element_type=jnp.float32)
    xr_ref[...] = jnp.dot(x, wr_ref[...], preferred_element_type=jnp.float32)


def _mm2(x, Wl, Wr):
    n, k = x.shape
    m = Wl.shape[1]
    blk = 1000
    grid = n // blk
    return pl.pallas_call(
        _mm2_body,
        grid=(grid,),
        in_specs=[
            pl.BlockSpec((blk, k), lambda i: (i, 0)),
            pl.BlockSpec((k, m), lambda i: (0, 0)),
            pl.BlockSpec((k, m), lambda i: (0, 0)),
        ],
        out_specs=[
            pl.BlockSpec((blk, m), lambda i: (i, 0)),
            pl.BlockSpec((blk, m), lambda i: (i, 0)),
        ],
        out_shape=[
            jax.ShapeDtypeStruct((n, m), jnp.float32),
            jax.ShapeDtypeStruct((n, m), jnp.float32),
        ],
    )(x, Wl, Wr)


def _gat_layer(x, src, dst, Wl, Wr, att, bias, heads, out_ch, concat=True):
    n = x.shape[0]
    xl, xr = _mm2(x, Wl, Wr)
    xl = xl.reshape(n, heads, out_ch)
    xr = xr.reshape(n, heads, out_ch)
    feat = xl[src] + xr[dst]
    e = jnp.sum(jax.nn.leaky_relu(feat, 0.2) * att[None], axis=-1)
    ex = jnp.exp(e)
    denom = jax.ops.segment_sum(ex, dst, num_segments=n)
    num = jax.ops.segment_sum(xl[src] * ex[:, :, None], dst, num_segments=n)
    out = num / (denom[:, :, None] + 1e-16)
    if concat:
        out = out.reshape(n, heads * out_ch)
    else:
        out = out.mean(axis=1)
    return out + bias


@jax.jit
def kernel(x, edge_index, Wl1, Wr1, att1, b1, Wl2, Wr2, att2, b2, Wl3, Wr3, att3, b3):
    loop = jnp.arange(N, dtype=edge_index.dtype)
    src = jnp.concatenate([edge_index[0], loop])
    dst = jnp.concatenate([edge_index[1], loop])
    h = _gat_layer(x, src, dst, Wl1, Wr1, att1, b1, HEADS, NHID)
    h = jax.nn.elu(h)
    h = _gat_layer(h, src, dst, Wl2, Wr2, att2, b2, HEADS, NHID)
    h = jax.nn.elu(h)
    h = _gat_layer(h, src, dst, Wl3, Wr3, att3, b3, 1, 40)
    return jax.nn.log_softmax(h, axis=-1)


# speedup vs baseline: 7.2871x; 6.3991x over previous
import functools

import jax
import jax.numpy as jnp
from jax import lax
from jax.experimental import pallas as pl
from jax.experimental.pallas import tpu as pltpu
from jax.experimental.pallas import tpu_sc as plsc

N = 10000
HEADS = 8
NHID = 16
NCLASS = 40
E_TOT = 320000 + N
CHUNK = 48
CPTQ = 116
QCAP = 16 * CPTQ * CHUNK
E_PART = 4 * QCAP
QS = 2528
QROWS = 2560
RPT = QROWS // 16
DPAD = 2540
DROWS = QROWS // 16
ROW = 128
DL3 = 48


def _make_sc_edge():
    mesh = plsc.VectorSubcoreMesh(core_axis_name="c", subcore_axis_name="s")

    @functools.partial(
        pl.kernel,
        out_type=(
            jax.ShapeDtypeStruct((4 * QROWS, ROW), jnp.float32),
            jax.ShapeDtypeStruct((4 * DROWS, ROW), jnp.float32),
        ),
        mesh=mesh,
        compiler_params=pltpu.CompilerParams(needs_layout_passes=False),
        scratch_types=[
            pltpu.VMEM((CHUNK,), jnp.int32),
            pltpu.VMEM((CHUNK,), jnp.int32),
            pltpu.VMEM((CHUNK,), jnp.int32),
            pltpu.VMEM((CHUNK,), jnp.int32),
            pltpu.VMEM((CHUNK,), jnp.int32),
            pltpu.VMEM((CHUNK,), jnp.int32),
            pltpu.VMEM((CHUNK,), jnp.int32),
            pltpu.VMEM((CHUNK,), jnp.int32),
            pltpu.VMEM((CHUNK,), jnp.int32),
            pltpu.VMEM((CHUNK,), jnp.int32),
            pltpu.VMEM((CHUNK, ROW), jnp.float32),
            pltpu.VMEM((CHUNK, ROW), jnp.float32),
            pltpu.VMEM((CHUNK, ROW), jnp.float32),
            pltpu.VMEM((CHUNK, ROW), jnp.float32),
            pltpu.VMEM((CHUNK, ROW), jnp.float32),
            pltpu.VMEM((HEADS * NHID * 16,), jnp.float32),
            pltpu.VMEM((HEADS * HEADS * 16,), jnp.float32),
            pltpu.VMEM_SHARED((QROWS, ROW), jnp.float32),
            pltpu.VMEM_SHARED((DROWS, ROW), jnp.float32),
            pltpu.SemaphoreType.DMA,
            pltpu.SemaphoreType.DMA,
            pltpu.SemaphoreType.DMA,
            pltpu.SemaphoreType.DMA,
            pltpu.SemaphoreType.DMA,
            pltpu.SemaphoreType.DMA,
        ],
    )
    def k(xl_hbm, xr_hbm, src_hbm, dst_hbm, att_hbm, m_hbm, zero_hbm,
          outf_hbm, outd_hbm,
          sidx0, didx0, sidx1, didx1, sidx2, didx2, sidx3, didx3,
          dridx, prevc,
          xlb0, xrb0, xlb1, xrb1, cbden, att_v, m_v,
          acc, dacc, gsem0, gsem1, isem0, isem1, isem2, isem3):
        cid = lax.axis_index("c")
        sid = lax.axis_index("s")
        r0 = pl.multiple_of(sid * RPT, 8)
        pltpu.sync_copy(att_hbm, att_v)
        pltpu.sync_copy(m_hbm, m_v)
        pltpu.sync_copy(zero_hbm.at[pl.ds(0, CHUNK)], cbden)
        lanes = jnp.arange(16, dtype=jnp.int32)
        zeros16 = jnp.zeros((16,), jnp.float32)
        for g in range(CHUNK // 16):
            prevc[pl.ds(g * 16, 16)] = jnp.zeros((16,), jnp.int32)

        sbuf = (sidx0, sidx1, sidx2, sidx3)
        dbuf = (didx0, didx1, didx2, didx3)
        xlbuf = (xlb0, xlb1)
        xrbuf = (xrb0, xrb1)
        gsem = (gsem0, gsem1)
        isem = (isem0, isem1, isem2, isem3)

        def phase_body(p, pcarry):
            qi = cid * 2 + p
            ebase = qi * QCAP + sid * (CPTQ * CHUNK)

            pltpu.sync_copy(zero_hbm.at[pl.ds(0, RPT)],
                            acc.at[pl.ds(r0, RPT)])

            @pl.when(sid == 0)
            def _():
                pltpu.sync_copy(zero_hbm.at[pl.ds(0, DROWS)], dacc)

            plsc.subcore_barrier()

            def load_idx(ci, i4, sync):
                off = pl.multiple_of(ebase + ci * CHUNK, CHUNK)
                if sync:
                    pltpu.sync_copy(src_hbm.at[pl.ds(off, CHUNK)], sbuf[i4])
                    pltpu.sync_copy(dst_hbm.at[pl.ds(off, CHUNK)], dbuf[i4])
                else:
                    pltpu.async_copy(src_hbm.at[pl.ds(off, CHUNK)],
                                     sbuf[i4], isem[i4])
                    pltpu.async_copy(dst_hbm.at[pl.ds(off, CHUNK)],
                                     dbuf[i4], isem[i4])

            def wait_idx(i4):
                pltpu.make_async_copy(src_hbm.at[pl.ds(0, CHUNK)],
                                      sbuf[i4], isem[i4]).wait()
                pltpu.make_async_copy(dst_hbm.at[pl.ds(0, CHUNK)],
                                      dbuf[i4], isem[i4]).wait()

            def fire_gather(i4, d2):
                pltpu.async_copy(xl_hbm.at[sbuf[i4]], xlbuf[d2], gsem[d2])
                pltpu.async_copy(xr_hbm.at[dbuf[i4]], xrbuf[d2], gsem[d2])

            def wait_gather(i4, d2):
                pltpu.make_async_copy(xl_hbm.at[sbuf[i4]],
                                      xlbuf[d2], gsem[d2]).wait()
                pltpu.make_async_copy(xr_hbm.at[dbuf[i4]],
                                      xrbuf[d2], gsem[d2]).wait()

            def compute(i4, d2):
                xlb = xlbuf[d2]
                xrb = xrbuf[d2]
                didx = dbuf[i4]

                def group_body(g, gcarry):
                    rows = lanes + g * 16
                    dstv = didx[pl.ds(g * 16, 16)]
                    dridx[pl.ds(g * 16, 16)] = lax.shift_right_logical(
                        dstv, 4)
                    dcol0 = (dstv & 15) * 8
                    prev = prevc[pl.ds(g * 16, 16)]
                    for h in range(HEADS):
                        plsc.store_scatter(cbden, [rows, prev + h], zeros16)
                    prevc[pl.ds(g * 16, 16)] = dcol0
                    evs = []
                    for h in range(HEADS):
                        e_acc = jnp.zeros((16,), jnp.float32)
                        for c in range(NHID):
                            col = jnp.full((16,), h * 16 + c, jnp.int32)
                            xlv = plsc.load_gather(xlb, [rows, col])
                            xrv = plsc.load_gather(xrb, [rows, col])
                            f = xlv + xrv
                            lr = 0.6 * f + 0.4 * jnp.abs(f)
                            attv = att_v[pl.ds((h * 16 + c) * 16, 16)]
                            e_acc = e_acc + lr * attv
                        evs.append(e_acc)
                    for h in range(HEADS):
                        combo = jnp.zeros((16,), jnp.float32)
                        for kk in range(HEADS):
                            mv = m_v[pl.ds((h * HEADS + kk) * 16, 16)]
                            combo = combo + evs[kk] * mv
                        ex = jnp.exp(combo)
                        for c in range(NHID):
                            col = jnp.full((16,), h * 16 + c, jnp.int32)
                            xlv = plsc.load_gather(xlb, [rows, col])
                            plsc.store_scatter(xrb, [rows, col], ex * xlv)
                        plsc.store_scatter(cbden, [rows, dcol0 + h], ex)
                    return gcarry

                lax.fori_loop(0, CHUNK // 16, group_body, 0)
                pltpu.sync_copy(xrb, acc.at[didx], add=True)
                pltpu.sync_copy(cbden, dacc.at[dridx], add=True)

            load_idx(0, 0, True)
            fire_gather(0, 0)
            load_idx(1, 1, True)

            def body(kk4, carry):
                base = kk4 * 4
                fire_gather(1, 1)
                load_idx(base + 2, 2, False)
                load_idx(base + 3, 3, False)
                wait_gather(0, 0)
                compute(0, 0)
                wait_idx(2)
                fire_gather(2, 0)
                wait_gather(1, 1)
                compute(1, 1)
                wait_idx(3)
                fire_gather(3, 1)

                @pl.when(base + 4 < CPTQ)
                def _():
                    load_idx(base + 4, 0, False)

                wait_gather(2, 0)
                compute(2, 0)

                @pl.when(base + 4 < CPTQ)
                def _():
                    wait_idx(0)
                    fire_gather(0, 0)

                @pl.when(base + 5 < CPTQ)
                def _():
                    load_idx(base + 5, 1, False)

                wait_gather(3, 1)
                compute(3, 1)

                @pl.when(base + 5 < CPTQ)
                def _():
                    wait_idx(1)

                return carry

            lax.fori_loop(0, CPTQ // 4, body, 0)

            plsc.subcore_barrier()
            pltpu.sync_copy(acc.at[pl.ds(r0, RPT)],
                            outf_hbm.at[pl.ds(qi * QROWS + r0, RPT)])

            @pl.when(sid == 0)
            def _():
                pltpu.sync_copy(dacc, outd_hbm.at[pl.ds(qi * DROWS, DROWS)])

            return pcarry

        lax.fori_loop(0, 2, phase_body, 0)

    return k


_sc_edge = _make_sc_edge()


_BLK = 1000


def _mm2_body(x_ref, wl_ref, wr_ref, xl_ref, xr_ref):
    x = x_ref[...]
    xl_ref[...] = jnp.dot(x, wl_ref[...], preferred_element_type=jnp.float32)
    xr_ref[...] = jnp.dot(x, wr_ref[...], preferred_element_type=jnp.float32)


def _mm2(x, Wl, Wr):
    n, kk = x.shape
    m = Wl.shape[1]
    return pl.pallas_call(
        _mm2_body,
        grid=(n // _BLK,),
        in_specs=[
            pl.BlockSpec((_BLK, kk), lambda i: (i, 0)),
            pl.BlockSpec((kk, m), lambda i: (0, 0)),
            pl.BlockSpec((kk, m), lambda i: (0, 0)),
        ],
        out_specs=[
            pl.BlockSpec((_BLK, m), lambda i: (i, 0)),
            pl.BlockSpec((_BLK, m), lambda i: (i, 0)),
        ],
        out_shape=[
            jax.ShapeDtypeStruct((n, m), jnp.float32),
            jax.ShapeDtypeStruct((n, m), jnp.float32),
        ],
    )(x, Wl, Wr)


def _combine_body(a_ref, d_ref, b_ref, wl_ref, wr_ref, e8_ref,
                  xl_ref, xr_ref):
    den_rep = jnp.dot(d_ref[...], e8_ref[...],
                      preferred_element_type=jnp.float32) + 1e-16
    h = a_ref[...] / den_rep + b_ref[...]
    h = jnp.where(h > 0, h, jnp.exp(jnp.minimum(h, 0.0)) - 1.0)
    xl_ref[...] = jnp.dot(h, wl_ref[...], preferred_element_type=jnp.float32)
    xr_ref[...] = jnp.dot(h, wr_ref[...], preferred_element_type=jnp.float32)


def _combine(num, den, b, Wl, Wr, e8):
    m = Wl.shape[1]
    return pl.pallas_call(
        _combine_body,
        grid=(N // _BLK,),
        in_specs=[
            pl.BlockSpec((_BLK, ROW), lambda i: (i, 0)),
            pl.BlockSpec((_BLK, HEADS), lambda i: (i, 0)),
            pl.BlockSpec((1, 128), lambda i: (0, 0)),
            pl.BlockSpec((128, m), lambda i: (0, 0)),
            pl.BlockSpec((128, m), lambda i: (0, 0)),
            pl.BlockSpec((HEADS, 128), lambda i: (0, 0)),
        ],
        out_specs=[
            pl.BlockSpec((_BLK, m), lambda i: (i, 0)),
            pl.BlockSpec((_BLK, m), lambda i: (i, 0)),
        ],
        out_shape=[
            jax.ShapeDtypeStruct((N, m), jnp.float32),
            jax.ShapeDtypeStruct((N, m), jnp.float32),
        ],
    )(num, den, b, Wl, Wr, e8)


def _finalize_body(a_ref, d_ref, b_ref, e8_ref, o_ref):
    den_rep = jnp.dot(d_ref[...], e8_ref[...],
                      preferred_element_type=jnp.float32) + 1e-16
    t = a_ref[...] / den_rep + b_ref[...]
    colid = lax.broadcasted_iota(jnp.int32, t.shape, 1)
    mask = colid < NCLASS
    tm = jnp.where(mask, t, -1e30)
    m = jnp.max(tm, axis=1, keepdims=True)
    z = jnp.where(mask, jnp.exp(tm - m), 0.0)
    lse = jnp.log(jnp.sum(z, axis=1, keepdims=True))
    o_ref[...] = t - m - lse


def _finalize(num, den, b3p, e8):
    return pl.pallas_call(
        _finalize_body,
        grid=(N // _BLK,),
        in_specs=[
            pl.BlockSpec((_BLK, ROW), lambda i: (i, 0)),
            pl.BlockSpec((_BLK, HEADS), lambda i: (i, 0)),
            pl.BlockSpec((1, ROW), lambda i: (0, 0)),
            pl.BlockSpec((HEADS, 128), lambda i: (0, 0)),
        ],
        out_specs=pl.BlockSpec((_BLK, ROW), lambda i: (i, 0)),
        out_shape=jax.ShapeDtypeStruct((N, ROW), jnp.float32),
    )(num, den, b3p, e8)


def _partition_edges(edge_index):
    loop = jnp.arange(N, dtype=edge_index.dtype)
    s_all = jnp.concatenate([edge_index[0], loop])
    d_all = jnp.concatenate([edge_index[1], loop])
    qi = d_all // QS
    d_local = d_all - qi * QS
    dest = jnp.zeros((E_TOT,), jnp.int32)
    for q in range(4):
        in_q = qi == q
        pos = jnp.cumsum(in_q.astype(jnp.int32)) - 1
        dest = jnp.where(in_q, q * QCAP + pos, dest)
    srcp = jnp.zeros((E_PART,), jnp.int32).at[dest].set(s_all)
    dstp = jnp.full((E_PART,), DPAD, jnp.int32).at[dest].set(d_local)
    return srcp, dstp


def _assemble(outf, outd):
    num = jnp.concatenate(
        [outf[q * QROWS:q * QROWS + QS] for q in range(4)])[:N]
    den = jnp.concatenate(
        [outd[q * DROWS:q * DROWS + QS // 16] for q in range(4)])
    den = den.reshape(-1, HEADS)[:N]
    return num, den


@jax.jit
def kernel(x, edge_index, Wl1, Wr1, att1, b1, Wl2, Wr2, att2, b2,
           Wl3, Wr3, att3, b3):
    srcp, dstp = _partition_edges(edge_index)
    zero = jnp.zeros((DROWS, ROW), jnp.float32)
    e8 = jnp.kron(jnp.eye(HEADS, dtype=jnp.float32),
                  jnp.ones((1, NHID), jnp.float32))

    def bcast16(v):
        return jnp.repeat(v.reshape(-1, 1), 16, axis=1).reshape(-1)

    att1b = bcast16(att1)
    att2b = bcast16(att2)
    m_eye = bcast16(jnp.eye(HEADS, dtype=jnp.float32))
    nh3 = DL3 // NHID
    m_l3 = bcast16(((jnp.arange(HEADS)[:, None] < nh3)
                    & (jnp.arange(HEADS)[None, :] < nh3))
                   .astype(jnp.float32))

    xl1, xr1 = _mm2(x, Wl1, Wr1)
    outf1, outd1 = _sc_edge(xl1, xr1, srcp, dstp, att1b, m_eye, zero)
    num1, den1 = _assemble(outf1, outd1)

    xl2, xr2 = _combine(num1, den1, b1.reshape(1, 128), Wl2, Wr2, e8)
    outf2, outd2 = _sc_edge(xl2, xr2, srcp, dstp, att2b, m_eye, zero)
    num2, den2 = _assemble(outf2, outd2)

    Wl3p = jnp.pad(Wl3, ((0, 0), (0, ROW - NCLASS)))
    Wr3p = jnp.pad(Wr3, ((0, 0), (0, ROW - NCLASS)))
    att3b = bcast16(jnp.pad(att3.reshape(-1), (0, ROW - NCLASS)))
    b3p = jnp.pad(b3, (0, ROW - NCLASS)).reshape(1, ROW)
    xl3, xr3 = _combine(num2, den2, b2.reshape(1, 128), Wl3p, Wr3p, e8)
    outf3, outd3 = _sc_edge(xl3, xr3, srcp, dstp, att3b, m_l3, zero)
    num3, den3 = _assemble(outf3, outd3)

    out = _finalize(num3, den3, b3p, e8)
    return out[:, :NCLASS]


# speedup vs baseline: 12.1598x; 1.6687x over previous
import functools

import jax
import jax.numpy as jnp
from jax import lax
from jax.experimental import pallas as pl
from jax.experimental.pallas import tpu as pltpu
from jax.experimental.pallas import tpu_sc as plsc

N = 10000
HEADS = 8
NHID = 16
NCLASS = 40
E_TOT = 320000 + N
CHUNK = 48
CPTQ = 116
QCAP = 16 * CPTQ * CHUNK
E_PART = 4 * QCAP
QS = 2528
QROWS = 2560
RPT = QROWS // 16
DPAD = 2540
DROWS = QROWS // 16
ROW = 128
DL3 = 48


def _make_sc_edge():
    mesh = plsc.VectorSubcoreMesh(core_axis_name="c", subcore_axis_name="s")

    @functools.partial(
        pl.kernel,
        out_type=(
            jax.ShapeDtypeStruct((4 * QROWS, ROW), jnp.float32),
            jax.ShapeDtypeStruct((4 * DROWS, ROW), jnp.float32),
        ),
        mesh=mesh,
        compiler_params=pltpu.CompilerParams(needs_layout_passes=False),
        scratch_types=[
            pltpu.VMEM((CHUNK,), jnp.int32),
            pltpu.VMEM((CHUNK,), jnp.int32),
            pltpu.VMEM((CHUNK,), jnp.int32),
            pltpu.VMEM((CHUNK,), jnp.int32),
            pltpu.VMEM((CHUNK,), jnp.int32),
            pltpu.VMEM((CHUNK,), jnp.int32),
            pltpu.VMEM((CHUNK,), jnp.int32),
            pltpu.VMEM((CHUNK,), jnp.int32),
            pltpu.VMEM((CHUNK,), jnp.int32),
            pltpu.VMEM((CHUNK,), jnp.int32),
            pltpu.VMEM((CHUNK, ROW), jnp.float32),
            pltpu.VMEM((CHUNK, ROW), jnp.float32),
            pltpu.VMEM((CHUNK, ROW), jnp.float32),
            pltpu.VMEM((CHUNK, ROW), jnp.float32),
            pltpu.VMEM((CHUNK, ROW), jnp.float32),
            pltpu.VMEM((HEADS * CHUNK,), jnp.float32),
            pltpu.VMEM((HEADS * NHID,), jnp.float32),
            pltpu.VMEM((HEADS * HEADS * 16,), jnp.float32),
            pltpu.VMEM_SHARED((QROWS, ROW), jnp.float32),
            pltpu.VMEM_SHARED((DROWS, ROW), jnp.float32),
            pltpu.SemaphoreType.DMA,
            pltpu.SemaphoreType.DMA,
            pltpu.SemaphoreType.DMA,
            pltpu.SemaphoreType.DMA,
            pltpu.SemaphoreType.DMA,
            pltpu.SemaphoreType.DMA,
        ],
    )
    def k(xl_hbm, xr_hbm, src_hbm, dst_hbm, att_hbm, m_hbm, zero_hbm,
          outf_hbm, outd_hbm,
          sidx0, didx0, sidx1, didx1, sidx2, didx2, sidx3, didx3,
          dridx, prevc,
          xlb0, xrb0, xlb1, xrb1, cbden, exbuf, att_v, m_v,
          acc, dacc, gsem0, gsem1, isem0, isem1, isem2, isem3):
        cid = lax.axis_index("c")
        sid = lax.axis_index("s")
        r0 = pl.multiple_of(sid * RPT, 8)
        pltpu.sync_copy(att_hbm, att_v)
        pltpu.sync_copy(m_hbm, m_v)
        pltpu.sync_copy(zero_hbm.at[pl.ds(0, CHUNK)], cbden)
        lanes = jnp.arange(16, dtype=jnp.int32)
        zeros16 = jnp.zeros((16,), jnp.float32)
        lane0 = lanes == 0
        ms = [[m_v[pl.ds((h * HEADS + kk) * 16, 16)][0]
               for kk in range(HEADS)] for h in range(HEADS)]
        for g in range(CHUNK // 16):
            prevc[pl.ds(g * 16, 16)] = jnp.zeros((16,), jnp.int32)

        sbuf = (sidx0, sidx1, sidx2, sidx3)
        dbuf = (didx0, didx1, didx2, didx3)
        xlbuf = (xlb0, xlb1)
        xrbuf = (xrb0, xrb1)
        gsem = (gsem0, gsem1)
        isem = (isem0, isem1, isem2, isem3)

        def phase_body(p, pcarry):
            qi = cid * 2 + p
            ebase = qi * QCAP + sid * (CPTQ * CHUNK)

            pltpu.sync_copy(zero_hbm.at[pl.ds(0, RPT)],
                            acc.at[pl.ds(r0, RPT)])

            @pl.when(sid == 0)
            def _():
                pltpu.sync_copy(zero_hbm.at[pl.ds(0, DROWS)], dacc)

            plsc.subcore_barrier()

            def load_idx(ci, i4, sync):
                off = pl.multiple_of(ebase + ci * CHUNK, CHUNK)
                if sync:
                    pltpu.sync_copy(src_hbm.at[pl.ds(off, CHUNK)], sbuf[i4])
                    pltpu.sync_copy(dst_hbm.at[pl.ds(off, CHUNK)], dbuf[i4])
                else:
                    pltpu.async_copy(src_hbm.at[pl.ds(off, CHUNK)],
                                     sbuf[i4], isem[i4])
                    pltpu.async_copy(dst_hbm.at[pl.ds(off, CHUNK)],
                                     dbuf[i4], isem[i4])

            def wait_idx(i4):
                pltpu.make_async_copy(src_hbm.at[pl.ds(0, CHUNK)],
                                      sbuf[i4], isem[i4]).wait()
                pltpu.make_async_copy(dst_hbm.at[pl.ds(0, CHUNK)],
                                      dbuf[i4], isem[i4]).wait()

            def fire_gather(i4, d2):
                pltpu.async_copy(xl_hbm.at[sbuf[i4]], xlbuf[d2], gsem[d2])
                pltpu.async_copy(xr_hbm.at[dbuf[i4]], xrbuf[d2], gsem[d2])

            def wait_gather(i4, d2):
                pltpu.make_async_copy(xl_hbm.at[sbuf[i4]],
                                      xlbuf[d2], gsem[d2]).wait()
                pltpu.make_async_copy(xr_hbm.at[dbuf[i4]],
                                      xrbuf[d2], gsem[d2]).wait()

            def compute(i4, d2):
                xlb = xlbuf[d2]
                xrb = xrbuf[d2]
                didx = dbuf[i4]

                def edge_body(e, ecarry):
                    xls = []
                    es = []
                    for h in range(HEADS):
                        xlrow = xlb[e, pl.ds(h * 16, 16)]
                        xrrow = xrb[e, pl.ds(h * 16, 16)]
                        f = xlrow + xrrow
                        lr = 0.6 * f + 0.4 * jnp.abs(f)
                        attv = att_v[pl.ds(h * 16, 16)]
                        es.append(jnp.sum(lr * attv))
                        xls.append(xlrow)
                    for h in range(HEADS):
                        combo = es[0] * ms[h][0]
                        for kk in range(1, HEADS):
                            combo = combo + es[kk] * ms[h][kk]
                        bex = jnp.exp(jnp.full((16,), combo, jnp.float32))
                        xrb[e, pl.ds(h * 16, 16)] = bex * xls[h]
                        plsc.store_scatter(
                            exbuf, [jnp.full((16,), h * CHUNK + e, jnp.int32)],
                            bex, mask=lane0)
                    return ecarry

                lax.fori_loop(0, CHUNK, edge_body, 0)

                for g in range(CHUNK // 16):
                    rows = lanes + g * 16
                    dstv = didx[pl.ds(g * 16, 16)]
                    dridx[pl.ds(g * 16, 16)] = lax.shift_right_logical(
                        dstv, 4)
                    dcol0 = (dstv & 15) * 8
                    prev = prevc[pl.ds(g * 16, 16)]
                    for h in range(HEADS):
                        plsc.store_scatter(cbden, [rows, prev + h], zeros16)
                    prevc[pl.ds(g * 16, 16)] = dcol0
                    for h in range(HEADS):
                        exv = exbuf[pl.ds(h * CHUNK + g * 16, 16)]
                        plsc.store_scatter(cbden, [rows, dcol0 + h], exv)
                pltpu.sync_copy(xrb, acc.at[didx], add=True)
                pltpu.sync_copy(cbden, dacc.at[dridx], add=True)

            load_idx(0, 0, True)
            fire_gather(0, 0)
            load_idx(1, 1, True)

            def body(kk4, carry):
                base = kk4 * 4
                fire_gather(1, 1)
                load_idx(base + 2, 2, False)
                load_idx(base + 3, 3, False)
                wait_gather(0, 0)
                compute(0, 0)
                wait_idx(2)
                fire_gather(2, 0)
                wait_gather(1, 1)
                compute(1, 1)
                wait_idx(3)
                fire_gather(3, 1)

                @pl.when(base + 4 < CPTQ)
                def _():
                    load_idx(base + 4, 0, False)

                wait_gather(2, 0)
                compute(2, 0)

                @pl.when(base + 4 < CPTQ)
                def _():
                    wait_idx(0)
                    fire_gather(0, 0)

                @pl.when(base + 5 < CPTQ)
                def _():
                    load_idx(base + 5, 1, False)

                wait_gather(3, 1)
                compute(3, 1)

                @pl.when(base + 5 < CPTQ)
                def _():
                    wait_idx(1)

                return carry

            lax.fori_loop(0, CPTQ // 4, body, 0)

            plsc.subcore_barrier()
            pltpu.sync_copy(acc.at[pl.ds(r0, RPT)],
                            outf_hbm.at[pl.ds(qi * QROWS + r0, RPT)])

            @pl.when(sid == 0)
            def _():
                pltpu.sync_copy(dacc, outd_hbm.at[pl.ds(qi * DROWS, DROWS)])

            return pcarry

        lax.fori_loop(0, 2, phase_body, 0)

    return k


_sc_edge = _make_sc_edge()


_BLK = 1000


def _mm2_body(x_ref, wl_ref, wr_ref, xl_ref, xr_ref):
    x = x_ref[...]
    xl_ref[...] = jnp.dot(x, wl_ref[...], preferred_element_type=jnp.float32)
    xr_ref[...] = jnp.dot(x, wr_ref[...], preferred_element_type=jnp.float32)


def _mm2(x, Wl, Wr):
    n, kk = x.shape
    m = Wl.shape[1]
    return pl.pallas_call(
        _mm2_body,
        grid=(n // _BLK,),
        in_specs=[
            pl.BlockSpec((_BLK, kk), lambda i: (i, 0)),
            pl.BlockSpec((kk, m), lambda i: (0, 0)),
            pl.BlockSpec((kk, m), lambda i: (0, 0)),
        ],
        out_specs=[
            pl.BlockSpec((_BLK, m), lambda i: (i, 0)),
            pl.BlockSpec((_BLK, m), lambda i: (i, 0)),
        ],
        out_shape=[
            jax.ShapeDtypeStruct((n, m), jnp.float32),
            jax.ShapeDtypeStruct((n, m), jnp.float32),
        ],
    )(x, Wl, Wr)


def _combine_body(a_ref, d_ref, b_ref, wl_ref, wr_ref, e8_ref,
                  xl_ref, xr_ref):
    den_rep = jnp.dot(d_ref[...], e8_ref[...],
                      preferred_element_type=jnp.float32) + 1e-16
    h = a_ref[...] / den_rep + b_ref[...]
    h = jnp.where(h > 0, h, jnp.exp(jnp.minimum(h, 0.0)) - 1.0)
    xl_ref[...] = jnp.dot(h, wl_ref[...], preferred_element_type=jnp.float32)
    xr_ref[...] = jnp.dot(h, wr_ref[...], preferred_element_type=jnp.float32)


def _combine(num, den, b, Wl, Wr, e8):
    m = Wl.shape[1]
    return pl.pallas_call(
        _combine_body,
        grid=(N // _BLK,),
        in_specs=[
            pl.BlockSpec((_BLK, ROW), lambda i: (i, 0)),
            pl.BlockSpec((_BLK, HEADS), lambda i: (i, 0)),
            pl.BlockSpec((1, 128), lambda i: (0, 0)),
            pl.BlockSpec((128, m), lambda i: (0, 0)),
            pl.BlockSpec((128, m), lambda i: (0, 0)),
            pl.BlockSpec((HEADS, 128), lambda i: (0, 0)),
        ],
        out_specs=[
            pl.BlockSpec((_BLK, m), lambda i: (i, 0)),
            pl.BlockSpec((_BLK, m), lambda i: (i, 0)),
        ],
        out_shape=[
            jax.ShapeDtypeStruct((N, m), jnp.float32),
            jax.ShapeDtypeStruct((N, m), jnp.float32),
        ],
    )(num, den, b, Wl, Wr, e8)


def _finalize_body(a_ref, d_ref, b_ref, e8_ref, o_ref):
    den_rep = jnp.dot(d_ref[...], e8_ref[...],
                      preferred_element_type=jnp.float32) + 1e-16
    t = a_ref[...] / den_rep + b_ref[...]
    colid = lax.broadcasted_iota(jnp.int32, t.shape, 1)
    mask = colid < NCLASS
    tm = jnp.where(mask, t, -1e30)
    m = jnp.max(tm, axis=1, keepdims=True)
    z = jnp.where(mask, jnp.exp(tm - m), 0.0)
    lse = jnp.log(jnp.sum(z, axis=1, keepdims=True))
    o_ref[...] = t - m - lse


def _finalize(num, den, b3p, e8):
    return pl.pallas_call(
        _finalize_body,
        grid=(N // _BLK,),
        in_specs=[
            pl.BlockSpec((_BLK, ROW), lambda i: (i, 0)),
            pl.BlockSpec((_BLK, HEADS), lambda i: (i, 0)),
            pl.BlockSpec((1, ROW), lambda i: (0, 0)),
            pl.BlockSpec((HEADS, 128), lambda i: (0, 0)),
        ],
        out_specs=pl.BlockSpec((_BLK, ROW), lambda i: (i, 0)),
        out_shape=jax.ShapeDtypeStruct((N, ROW), jnp.float32),
    )(num, den, b3p, e8)


def _partition_edges(edge_index):
    loop = jnp.arange(N, dtype=edge_index.dtype)
    s_all = jnp.concatenate([edge_index[0], loop])
    d_all = jnp.concatenate([edge_index[1], loop])
    qi = d_all // QS
    d_local = d_all - qi * QS
    dest = jnp.zeros((E_TOT,), jnp.int32)
    for q in range(4):
        in_q = qi == q
        pos = jnp.cumsum(in_q.astype(jnp.int32)) - 1
        dest = jnp.where(in_q, q * QCAP + pos, dest)
    srcp = jnp.zeros((E_PART,), jnp.int32).at[dest].set(s_all)
    dstp = jnp.full((E_PART,), DPAD, jnp.int32).at[dest].set(d_local)
    return srcp, dstp


def _assemble(outf, outd):
    num = jnp.concatenate(
        [outf[q * QROWS:q * QROWS + QS] for q in range(4)])[:N]
    den = jnp.concatenate(
        [outd[q * DROWS:q * DROWS + QS // 16] for q in range(4)])
    den = den.reshape(-1, HEADS)[:N]
    return num, den


@jax.jit
def kernel(x, edge_index, Wl1, Wr1, att1, b1, Wl2, Wr2, att2, b2,
           Wl3, Wr3, att3, b3):
    srcp, dstp = _partition_edges(edge_index)
    zero = jnp.zeros((DROWS, ROW), jnp.float32)
    e8 = jnp.kron(jnp.eye(HEADS, dtype=jnp.float32),
                  jnp.ones((1, NHID), jnp.float32))

    def bcast16(v):
        return jnp.repeat(v.reshape(-1, 1), 16, axis=1).reshape(-1)

    att1b = att1.reshape(-1)
    att2b = att2.reshape(-1)
    m_eye = bcast16(jnp.eye(HEADS, dtype=jnp.float32))
    nh3 = DL3 // NHID
    m_l3 = bcast16(((jnp.arange(HEADS)[:, None] < nh3)
                    & (jnp.arange(HEADS)[None, :] < nh3))
                   .astype(jnp.float32))

    xl1, xr1 = _mm2(x, Wl1, Wr1)
    outf1, outd1 = _sc_edge(xl1, xr1, srcp, dstp, att1b, m_eye, zero)
    num1, den1 = _assemble(outf1, outd1)

    xl2, xr2 = _combine(num1, den1, b1.reshape(1, 128), Wl2, Wr2, e8)
    outf2, outd2 = _sc_edge(xl2, xr2, srcp, dstp, att2b, m_eye, zero)
    num2, den2 = _assemble(outf2, outd2)

    Wl3p = jnp.pad(Wl3, ((0, 0), (0, ROW - NCLASS)))
    Wr3p = jnp.pad(Wr3, ((0, 0), (0, ROW - NCLASS)))
    att3b = jnp.pad(att3.reshape(-1), (0, ROW - NCLASS))
    b3p = jnp.pad(b3, (0, ROW - NCLASS)).reshape(1, ROW)
    xl3, xr3 = _combine(num2, den2, b2.reshape(1, 128), Wl3p, Wr3p, e8)
    outf3, outd3 = _sc_edge(xl3, xr3, srcp, dstp, att3b, m_l3, zero)
    num3, den3 = _assemble(outf3, outd3)

    out = _finalize(num3, den3, b3p, e8)
    return out[:, :NCLASS]


# speedup vs baseline: 12.9589x; 1.0657x over previous
import functools

import jax
import jax.numpy as jnp
from jax import lax
from jax.experimental import pallas as pl
from jax.experimental.pallas import tpu as pltpu
from jax.experimental.pallas import tpu_sc as plsc

N = 10000
HEADS = 8
NHID = 16
NCLASS = 40
E_TOT = 320000 + N
CHUNK = 48
CPTQ = 116
QCAP = 16 * CPTQ * CHUNK
E_PART = 4 * QCAP
QS = 2528
QROWS = 2560
RPT = QROWS // 16
DPAD = 2540
DROWS = QROWS // 16
ROW = 128
DL3 = 48


def _make_sc_edge():
    mesh = plsc.VectorSubcoreMesh(core_axis_name="c", subcore_axis_name="s")

    @functools.partial(
        pl.kernel,
        out_type=(
            jax.ShapeDtypeStruct((4 * QROWS, ROW), jnp.float32),
            jax.ShapeDtypeStruct((4 * DROWS, ROW), jnp.float32),
        ),
        mesh=mesh,
        compiler_params=pltpu.CompilerParams(needs_layout_passes=False),
        scratch_types=[
            pltpu.VMEM((CHUNK,), jnp.int32),
            pltpu.VMEM((CHUNK,), jnp.int32),
            pltpu.VMEM((CHUNK,), jnp.int32),
            pltpu.VMEM((CHUNK,), jnp.int32),
            pltpu.VMEM((CHUNK,), jnp.int32),
            pltpu.VMEM((CHUNK,), jnp.int32),
            pltpu.VMEM((CHUNK,), jnp.int32),
            pltpu.VMEM((CHUNK,), jnp.int32),
            pltpu.VMEM((CHUNK,), jnp.int32),
            pltpu.VMEM((CHUNK,), jnp.int32),
            pltpu.VMEM((CHUNK, ROW), jnp.float32),
            pltpu.VMEM((CHUNK, ROW), jnp.float32),
            pltpu.VMEM((CHUNK, ROW), jnp.float32),
            pltpu.VMEM((CHUNK, ROW), jnp.float32),
            pltpu.VMEM((CHUNK, ROW), jnp.float32),
            pltpu.VMEM((HEADS * CHUNK,), jnp.float32),
            pltpu.VMEM((HEADS * NHID,), jnp.float32),
            pltpu.VMEM((HEADS * HEADS * 16,), jnp.float32),
            pltpu.VMEM_SHARED((QROWS, ROW), jnp.float32),
            pltpu.VMEM_SHARED((DROWS, ROW), jnp.float32),
            pltpu.SemaphoreType.DMA,
            pltpu.SemaphoreType.DMA,
            pltpu.SemaphoreType.DMA,
            pltpu.SemaphoreType.DMA,
            pltpu.SemaphoreType.DMA,
            pltpu.SemaphoreType.DMA,
        ],
    )
    def k(xl_hbm, xr_hbm, src_hbm, dst_hbm, att_hbm, m_hbm, zero_hbm,
          outf_hbm, outd_hbm,
          sidx0, didx0, sidx1, didx1, sidx2, didx2, sidx3, didx3,
          dridx, prevc,
          xlb0, xrb0, xlb1, xrb1, cbden, exbuf, att_v, m_v,
          acc, dacc, gsem0, gsem1, isem0, isem1, isem2, isem3):
        cid = lax.axis_index("c")
        sid = lax.axis_index("s")
        r0 = pl.multiple_of(sid * RPT, 8)
        pltpu.sync_copy(att_hbm, att_v)
        pltpu.sync_copy(m_hbm, m_v)
        pltpu.sync_copy(zero_hbm.at[pl.ds(0, CHUNK)], cbden)
        lanes = jnp.arange(16, dtype=jnp.int32)
        zeros16 = jnp.zeros((16,), jnp.float32)
        lane0 = lanes == 0
        ms = [[m_v[pl.ds((h * HEADS + kk) * 16, 16)][0]
               for kk in range(HEADS)] for h in range(HEADS)]
        for g in range(CHUNK // 16):
            prevc[pl.ds(g * 16, 16)] = jnp.zeros((16,), jnp.int32)

        sbuf = (sidx0, sidx1, sidx2, sidx3)
        dbuf = (didx0, didx1, didx2, didx3)
        xlbuf = (xlb0, xlb1)
        xrbuf = (xrb0, xrb1)
        gsem = (gsem0, gsem1)
        isem = (isem0, isem1, isem2, isem3)

        def phase_body(p, pcarry):
            qi = cid * 2 + p
            ebase = qi * QCAP + sid * (CPTQ * CHUNK)

            pltpu.sync_copy(zero_hbm.at[pl.ds(0, RPT)],
                            acc.at[pl.ds(r0, RPT)])

            @pl.when(sid == 0)
            def _():
                pltpu.sync_copy(zero_hbm.at[pl.ds(0, DROWS)], dacc)

            plsc.subcore_barrier()

            def load_idx(ci, i4, sync):
                off = pl.multiple_of(ebase + ci * CHUNK, CHUNK)
                if sync:
                    pltpu.sync_copy(src_hbm.at[pl.ds(off, CHUNK)], sbuf[i4])
                    pltpu.sync_copy(dst_hbm.at[pl.ds(off, CHUNK)], dbuf[i4])
                else:
                    pltpu.async_copy(src_hbm.at[pl.ds(off, CHUNK)],
                                     sbuf[i4], isem[i4])
                    pltpu.async_copy(dst_hbm.at[pl.ds(off, CHUNK)],
                                     dbuf[i4], isem[i4])

            def wait_idx(i4):
                pltpu.make_async_copy(src_hbm.at[pl.ds(0, CHUNK)],
                                      sbuf[i4], isem[i4]).wait()
                pltpu.make_async_copy(dst_hbm.at[pl.ds(0, CHUNK)],
                                      dbuf[i4], isem[i4]).wait()

            def fire_gather(i4, d2):
                pltpu.async_copy(xl_hbm.at[sbuf[i4]], xlbuf[d2], gsem[d2])
                pltpu.async_copy(xr_hbm.at[dbuf[i4]], xrbuf[d2], gsem[d2])

            def wait_gather(i4, d2):
                pltpu.make_async_copy(xl_hbm.at[sbuf[i4]],
                                      xlbuf[d2], gsem[d2]).wait()
                pltpu.make_async_copy(xr_hbm.at[dbuf[i4]],
                                      xrbuf[d2], gsem[d2]).wait()

            def compute(i4, d2):
                xlb = xlbuf[d2]
                xrb = xrbuf[d2]
                didx = dbuf[i4]

                def edge_body(e4, ecarry):
                    for u in range(4):
                        e = e4 * 4 + u
                        xls = []
                        es = []
                        for h in range(HEADS):
                            xlrow = xlb[e, pl.ds(h * 16, 16)]
                            xrrow = xrb[e, pl.ds(h * 16, 16)]
                            f = xlrow + xrrow
                            lr = 0.6 * f + 0.4 * jnp.abs(f)
                            attv = att_v[pl.ds(h * 16, 16)]
                            es.append(jnp.sum(lr * attv))
                            xls.append(xlrow)
                        for h in range(HEADS):
                            combo = es[0] * ms[h][0]
                            for kk in range(1, HEADS):
                                combo = combo + es[kk] * ms[h][kk]
                            bex = jnp.exp(
                                jnp.full((16,), combo, jnp.float32))
                            xrb[e, pl.ds(h * 16, 16)] = bex * xls[h]
                            plsc.store_scatter(
                                exbuf,
                                [jnp.full((16,), h * CHUNK + e, jnp.int32)],
                                bex, mask=lane0)
                    return ecarry

                lax.fori_loop(0, CHUNK // 4, edge_body, 0)

                for g in range(CHUNK // 16):
                    rows = lanes + g * 16
                    dstv = didx[pl.ds(g * 16, 16)]
                    dridx[pl.ds(g * 16, 16)] = lax.shift_right_logical(
                        dstv, 4)
                    dcol0 = (dstv & 15) * 8
                    prev = prevc[pl.ds(g * 16, 16)]
                    for h in range(HEADS):
                        plsc.store_scatter(cbden, [rows, prev + h], zeros16)
                    prevc[pl.ds(g * 16, 16)] = dcol0
                    for h in range(HEADS):
                        exv = exbuf[pl.ds(h * CHUNK + g * 16, 16)]
                        plsc.store_scatter(cbden, [rows, dcol0 + h], exv)
                pltpu.sync_copy(xrb, acc.at[didx], add=True)
                pltpu.sync_copy(cbden, dacc.at[dridx], add=True)

            load_idx(0, 0, True)
            fire_gather(0, 0)
            load_idx(1, 1, True)

            def body(kk4, carry):
                base = kk4 * 4
                fire_gather(1, 1)
                load_idx(base + 2, 2, False)
                load_idx(base + 3, 3, False)
                wait_gather(0, 0)
                compute(0, 0)
                wait_idx(2)
                fire_gather(2, 0)
                wait_gather(1, 1)
                compute(1, 1)
                wait_idx(3)
                fire_gather(3, 1)

                @pl.when(base + 4 < CPTQ)
                def _():
                    load_idx(base + 4, 0, False)

                wait_gather(2, 0)
                compute(2, 0)

                @pl.when(base + 4 < CPTQ)
                def _():
                    wait_idx(0)
                    fire_gather(0, 0)

                @pl.when(base + 5 < CPTQ)
                def _():
                    load_idx(base + 5, 1, False)

                wait_gather(3, 1)
                compute(3, 1)

                @pl.when(base + 5 < CPTQ)
                def _():
                    wait_idx(1)

                return carry

            lax.fori_loop(0, CPTQ // 4, body, 0)

            plsc.subcore_barrier()
            pltpu.sync_copy(acc.at[pl.ds(r0, RPT)],
                            outf_hbm.at[pl.ds(qi * QROWS + r0, RPT)])

            @pl.when(sid == 0)
            def _():
                pltpu.sync_copy(dacc, outd_hbm.at[pl.ds(qi * DROWS, DROWS)])

            return pcarry

        lax.fori_loop(0, 2, phase_body, 0)

    return k


_sc_edge = _make_sc_edge()


_BLK = 1000


def _mm2_body(x_ref, wl_ref, wr_ref, xl_ref, xr_ref):
    x = x_ref[...]
    xl_ref[...] = jnp.dot(x, wl_ref[...], preferred_element_type=jnp.float32)
    xr_ref[...] = jnp.dot(x, wr_ref[...], preferred_element_type=jnp.float32)


def _mm2(x, Wl, Wr):
    n, kk = x.shape
    m = Wl.shape[1]
    return pl.pallas_call(
        _mm2_body,
        grid=(n // _BLK,),
        in_specs=[
            pl.BlockSpec((_BLK, kk), lambda i: (i, 0)),
            pl.BlockSpec((kk, m), lambda i: (0, 0)),
            pl.BlockSpec((kk, m), lambda i: (0, 0)),
        ],
        out_specs=[
            pl.BlockSpec((_BLK, m), lambda i: (i, 0)),
            pl.BlockSpec((_BLK, m), lambda i: (i, 0)),
        ],
        out_shape=[
            jax.ShapeDtypeStruct((n, m), jnp.float32),
            jax.ShapeDtypeStruct((n, m), jnp.float32),
        ],
    )(x, Wl, Wr)


def _combine_body(a_ref, d_ref, b_ref, wl_ref, wr_ref, e8_ref,
                  xl_ref, xr_ref):
    den_rep = jnp.dot(d_ref[...], e8_ref[...],
                      preferred_element_type=jnp.float32) + 1e-16
    h = a_ref[...] / den_rep + b_ref[...]
    h = jnp.where(h > 0, h, jnp.exp(jnp.minimum(h, 0.0)) - 1.0)
    xl_ref[...] = jnp.dot(h, wl_ref[...], preferred_element_type=jnp.float32)
    xr_ref[...] = jnp.dot(h, wr_ref[...], preferred_element_type=jnp.float32)


def _combine(num, den, b, Wl, Wr, e8):
    m = Wl.shape[1]
    return pl.pallas_call(
        _combine_body,
        grid=(N // _BLK,),
        in_specs=[
            pl.BlockSpec((_BLK, ROW), lambda i: (i, 0)),
            pl.BlockSpec((_BLK, HEADS), lambda i: (i, 0)),
            pl.BlockSpec((1, 128), lambda i: (0, 0)),
            pl.BlockSpec((128, m), lambda i: (0, 0)),
            pl.BlockSpec((128, m), lambda i: (0, 0)),
            pl.BlockSpec((HEADS, 128), lambda i: (0, 0)),
        ],
        out_specs=[
            pl.BlockSpec((_BLK, m), lambda i: (i, 0)),
            pl.BlockSpec((_BLK, m), lambda i: (i, 0)),
        ],
        out_shape=[
            jax.ShapeDtypeStruct((N, m), jnp.float32),
            jax.ShapeDtypeStruct((N, m), jnp.float32),
        ],
    )(num, den, b, Wl, Wr, e8)


def _finalize_body(a_ref, d_ref, b_ref, e8_ref, o_ref):
    den_rep = jnp.dot(d_ref[...], e8_ref[...],
                      preferred_element_type=jnp.float32) + 1e-16
    t = a_ref[...] / den_rep + b_ref[...]
    colid = lax.broadcasted_iota(jnp.int32, t.shape, 1)
    mask = colid < NCLASS
    tm = jnp.where(mask, t, -1e30)
    m = jnp.max(tm, axis=1, keepdims=True)
    z = jnp.where(mask, jnp.exp(tm - m), 0.0)
    lse = jnp.log(jnp.sum(z, axis=1, keepdims=True))
    o_ref[...] = t - m - lse


def _finalize(num, den, b3p, e8):
    return pl.pallas_call(
        _finalize_body,
        grid=(N // _BLK,),
        in_specs=[
            pl.BlockSpec((_BLK, ROW), lambda i: (i, 0)),
            pl.BlockSpec((_BLK, HEADS), lambda i: (i, 0)),
            pl.BlockSpec((1, ROW), lambda i: (0, 0)),
            pl.BlockSpec((HEADS, 128), lambda i: (0, 0)),
        ],
        out_specs=pl.BlockSpec((_BLK, ROW), lambda i: (i, 0)),
        out_shape=jax.ShapeDtypeStruct((N, ROW), jnp.float32),
    )(num, den, b3p, e8)


def _partition_edges(edge_index):
    loop = jnp.arange(N, dtype=edge_index.dtype)
    s_all = jnp.concatenate([edge_index[0], loop])
    d_all = jnp.concatenate([edge_index[1], loop])
    qi = d_all // QS
    d_local = d_all - qi * QS
    dest = jnp.zeros((E_TOT,), jnp.int32)
    for q in range(4):
        in_q = qi == q
        pos = jnp.cumsum(in_q.astype(jnp.int32)) - 1
        dest = jnp.where(in_q, q * QCAP + pos, dest)
    srcp = jnp.zeros((E_PART,), jnp.int32).at[dest].set(s_all)
    dstp = jnp.full((E_PART,), DPAD, jnp.int32).at[dest].set(d_local)
    return srcp, dstp


def _assemble(outf, outd):
    num = jnp.concatenate(
        [outf[q * QROWS:q * QROWS + QS] for q in range(4)])[:N]
    den = jnp.concatenate(
        [outd[q * DROWS:q * DROWS + QS // 16] for q in range(4)])
    den = den.reshape(-1, HEADS)[:N]
    return num, den


@jax.jit
def kernel(x, edge_index, Wl1, Wr1, att1, b1, Wl2, Wr2, att2, b2,
           Wl3, Wr3, att3, b3):
    srcp, dstp = _partition_edges(edge_index)
    zero = jnp.zeros((DROWS, ROW), jnp.float32)
    e8 = jnp.kron(jnp.eye(HEADS, dtype=jnp.float32),
                  jnp.ones((1, NHID), jnp.float32))

    def bcast16(v):
        return jnp.repeat(v.reshape(-1, 1), 16, axis=1).reshape(-1)

    att1b = att1.reshape(-1)
    att2b = att2.reshape(-1)
    m_eye = bcast16(jnp.eye(HEADS, dtype=jnp.float32))
    nh3 = DL3 // NHID
    m_l3 = bcast16(((jnp.arange(HEADS)[:, None] < nh3)
                    & (jnp.arange(HEADS)[None, :] < nh3))
                   .astype(jnp.float32))

    xl1, xr1 = _mm2(x, Wl1, Wr1)
    outf1, outd1 = _sc_edge(xl1, xr1, srcp, dstp, att1b, m_eye, zero)
    num1, den1 = _assemble(outf1, outd1)

    xl2, xr2 = _combine(num1, den1, b1.reshape(1, 128), Wl2, Wr2, e8)
    outf2, outd2 = _sc_edge(xl2, xr2, srcp, dstp, att2b, m_eye, zero)
    num2, den2 = _assemble(outf2, outd2)

    Wl3p = jnp.pad(Wl3, ((0, 0), (0, ROW - NCLASS)))
    Wr3p = jnp.pad(Wr3, ((0, 0), (0, ROW - NCLASS)))
    att3b = jnp.pad(att3.reshape(-1), (0, ROW - NCLASS))
    b3p = jnp.pad(b3, (0, ROW - NCLASS)).reshape(1, ROW)
    xl3, xr3 = _combine(num2, den2, b2.reshape(1, 128), Wl3p, Wr3p, e8)
    outf3, outd3 = _sc_edge(xl3, xr3, srcp, dstp, att3b, m_l3, zero)
    num3, den3 = _assemble(outf3, outd3)

    out = _finalize(num3, den3, b3p, e8)
    return out[:, :NCLASS]
